# Initial kernel scaffold; baseline (speedup 1.0000x reference)
import jax
import jax.numpy as jnp
from jax.experimental import pallas as pl


def kernel(x, pos_embedding):
    raise NotImplementedError("write your pallas kernel here")


# speedup vs baseline: 5.0576x; 5.0576x over previous
import functools

import jax
import jax.numpy as jnp
from jax import lax
from jax.experimental import pallas as pl
from jax.experimental.pallas import tpu as pltpu
from jax.experimental.pallas import tpu_sc as plsc

S_LEN = 8192
BATCH = 4
D_MODEL = 1024

_info = plsc.get_sparse_core_info()
_NC, _NS = _info.num_cores, _info.num_subcores
_NW = _NC * _NS

_ROWS_PER_W = S_LEN // _NW
_CH = 32
_NCHUNK = _ROWS_PER_W // _CH


def _body(emb_hbm, out_hbm, buf, sem):
    wid = lax.axis_index("s") * _NC + lax.axis_index("c")
    base = wid * _ROWS_PER_W
    for g in range(_NCHUNK):
        b = base + g * _CH
        pltpu.async_copy(emb_hbm.at[pl.ds(b, _CH)], buf, sem).wait()
        for n in range(BATCH):
            pltpu.sync_copy(buf, out_hbm.at[pl.ds(b, _CH), n])


@jax.jit
def _pos_broadcast(pos_embedding):
    mesh = plsc.VectorSubcoreMesh(core_axis_name="c", subcore_axis_name="s")
    return pl.kernel(
        _body,
        out_type=jax.ShapeDtypeStruct((S_LEN, BATCH, D_MODEL), jnp.float32),
        mesh=mesh,
        scratch_types=[
            pltpu.VMEM((_CH, D_MODEL), jnp.float32),
            pltpu.SemaphoreType.DMA,
        ],
    )(pos_embedding)


def kernel(x, pos_embedding):
    del x
    return _pos_broadcast(pos_embedding)


# speedup vs baseline: 5.2567x; 1.0394x over previous
import functools

import jax
import jax.numpy as jnp
from jax import lax
from jax.experimental import pallas as pl
from jax.experimental.pallas import tpu as pltpu
from jax.experimental.pallas import tpu_sc as plsc

S_LEN = 8192
BATCH = 4
D_MODEL = 1024

_info = plsc.get_sparse_core_info()
_NC, _NS = _info.num_cores, _info.num_subcores
_NW = _NC * _NS

_ROWS_PER_W = S_LEN // _NW
_CH = 32
_NCHUNK = _ROWS_PER_W // _CH


def _body(emb_hbm, out_hbm, buf0, buf1, rs0, rs1, ws0, ws1):
    bufs, rsems, wsems = (buf0, buf1), (rs0, rs1), (ws0, ws1)
    wid = lax.axis_index("s") * _NC + lax.axis_index("c")
    base = wid * _ROWS_PER_W

    def read(g, sl):
        return pltpu.async_copy(
            emb_hbm.at[pl.ds(base + g * _CH, _CH)], bufs[sl], rsems[sl])

    rh = [read(0, 0), read(1, 1)]
    wh = [None, None]
    for g in range(_NCHUNK):
        sl = g & 1
        rh[sl].wait()
        b = base + g * _CH
        wh[sl] = [
            pltpu.async_copy(bufs[sl], out_hbm.at[pl.ds(b, _CH), n], wsems[sl])
            for n in range(BATCH)
        ]
        if g + 2 < _NCHUNK:
            for h in wh[sl]:
                h.wait()
            rh[sl] = read(g + 2, sl)
    for sl in range(2):
        for h in wh[sl]:
            h.wait()


@jax.jit
def _pos_broadcast(pos_embedding):
    mesh = plsc.VectorSubcoreMesh(core_axis_name="c", subcore_axis_name="s")
    return pl.kernel(
        _body,
        out_type=jax.ShapeDtypeStruct((S_LEN, BATCH, D_MODEL), jnp.float32),
        mesh=mesh,
        scratch_types=[
            pltpu.VMEM((_CH, D_MODEL), jnp.float32),
            pltpu.VMEM((_CH, D_MODEL), jnp.float32),
            pltpu.SemaphoreType.DMA,
            pltpu.SemaphoreType.DMA,
            pltpu.SemaphoreType.DMA,
            pltpu.SemaphoreType.DMA,
        ],
    )(pos_embedding)


def kernel(x, pos_embedding):
    del x
    return _pos_broadcast(pos_embedding)
